# baseline (speedup 1.0000x reference)
import jax
import jax.numpy as jnp
from jax import lax
from jax.experimental import pallas as pl
from jax.experimental.pallas import tpu as pltpu
from jax.experimental.pallas import tpu_sc as plsc

N = 10000
E = 320000
D = 32
DE = 16
H = 64
K = 4

NC = 2
NS = 16
NPAD = 10240
ROWS_PER_SUB = NPAD // NS
ACC_W = 128
TW = 128
EB = 80
EDGES_PER_SUB = E // NS
NBATCH = EDGES_PER_SUB // EB


def _sc_body(tbl, srcA, dstA, eaA, q2A, out,
             idx0, dst0, ea0, q20, g0, c0, ds0,
             idx1, dst1, ea1, q21, g1, c1, ds1,
             acc, sl0, sg0, ss0, sl1, sg1, ss1):
    c = lax.axis_index("c")
    s = lax.axis_index("s")
    set0 = (idx0, dst0, ea0, q20, g0, c0, ds0, sl0, sg0, ss0)
    set1 = (idx1, dst1, ea1, q21, g1, c1, ds1, sl1, sg1, ss1)

    zf = jnp.zeros((16,), jnp.float32)
    zi = jnp.zeros((16,), jnp.int32)

    def zrow_body(i, cc):
        for col in range(0, ACC_W, 16):
            c0[i, pl.ds(col, 16)] = zf
            c1[i, pl.ds(col, 16)] = zf
        return cc

    lax.fori_loop(0, EB, zrow_body, 0)
    for j in range(EB // 16):
        ds0[pl.ds(j * 16, 16)] = zi
        ds1[pl.ds(j * 16, 16)] = zi
    rbase = s * ROWS_PER_SUB
    for i in range(ROWS_PER_SUB // EB):
        pltpu.sync_copy(c0, acc.at[pl.ds(rbase + i * EB, EB)])
    plsc.subcore_barrier()

    ebase = s * EDGES_PER_SUB

    def issue_linear(b, S):
        idx_v, dst_v, ea_v, q2_v = S[0], S[1], S[2], S[3]
        sem_l = S[7]
        base = ebase + b * EB
        pltpu.async_copy(srcA.at[pl.ds(base, EB)], idx_v, sem_l)
        pltpu.async_copy(dstA.at[pl.ds(base, EB)], dst_v, sem_l)
        pltpu.async_copy(eaA.at[pl.ds(base, EB)], ea_v, sem_l)
        qbase = s * (EDGES_PER_SUB // 8) + b * (EB // 8)
        pltpu.async_copy(q2A.at[c, pl.ds(qbase, EB // 8)], q2_v, sem_l)

    def wait_linear(S):
        idx_v, dst_v, ea_v, q2_v = S[0], S[1], S[2], S[3]
        sem_l = S[7]
        pltpu.make_async_copy(srcA.at[pl.ds(0, EB)], idx_v, sem_l).wait()
        pltpu.make_async_copy(dstA.at[pl.ds(0, EB)], dst_v, sem_l).wait()
        pltpu.make_async_copy(eaA.at[pl.ds(0, EB)], ea_v, sem_l).wait()
        pltpu.make_async_copy(q2A.at[c, pl.ds(0, EB // 8)], q2_v,
                              sem_l).wait()

    def shift_and_gather(S):
        idx_v, g_v, sem_g = S[0], S[4], S[8]
        coff_v = jnp.full((16,), c * N, jnp.int32)
        for j in range(EB // 16):
            sl = pl.ds(j * 16, 16)
            idx_v[sl] = idx_v[sl] + coff_v
        pltpu.async_copy(tbl.at[idx_v], g_v, sem_g)

    def wait_gather(S):
        idx_v, g_v, sem_g = S[0], S[4], S[8]
        pltpu.make_async_copy(tbl.at[idx_v], g_v, sem_g).wait()

    def drain_scatter(S):
        c_v, dstS, sem_s = S[5], S[6], S[9]
        pltpu.make_async_copy(c_v, acc.at[dstS], sem_s).wait()

    def start_scatter(S):
        dst_v, c_v, dstS, sem_s = S[1], S[5], S[6], S[9]
        for j in range(EB // 16):
            sl = pl.ds(j * 16, 16)
            dstS[sl] = dst_v[sl]
        pltpu.async_copy(c_v, acc.at[dstS], sem_s, add=True)

    def compute(S):
        ea_v, q2_v, g_v, c_v = S[2], S[3], S[4], S[5]

        @plsc.parallel_loop(0, EB, 1, unroll=2)
        def edge_body(e):
            lanesf = jnp.arange(16, dtype=jnp.int32).astype(jnp.float32)
            m01f = jnp.maximum(0.0, jnp.minimum(1.0, 2.0 - lanesf))
            oh2c = jnp.maximum(0.0, 1.0 - jnp.abs(lanesf - 2.0))
            t = (g_v[e, pl.ds(64, 16)]
                 + q2_v[lax.shift_right_logical(e, 3),
                        pl.ds(lax.bitwise_and(e, 7) * 16, 16)])
            e2 = jnp.exp(t + t)
            ar = 1.0 - 2.0 / (e2 + 1.0)
            a0 = ar[0]
            a1 = ar[1]
            c_v[e, pl.ds(0, 16)] = a0 * g_v[e, pl.ds(0, 16)]
            c_v[e, pl.ds(16, 16)] = a0 * g_v[e, pl.ds(16, 16)]
            c_v[e, pl.ds(32, 16)] = a1 * g_v[e, pl.ds(32, 16)]
            c_v[e, pl.ds(48, 16)] = a1 * g_v[e, pl.ds(48, 16)]
            ear = ea_v[e, :]
            c_v[e, pl.ds(64, 16)] = a0 * ear
            c_v[e, pl.ds(80, 16)] = a1 * ear
            c_v[e, pl.ds(96, 16)] = ar * m01f + oh2c

    pltpu.async_copy(c0, acc.at[ds0], ss0, add=True)
    pltpu.async_copy(c1, acc.at[ds1], ss1, add=True)
    issue_linear(0, set0)
    issue_linear(1, set1)

    def pair_body(g, carry):
        b0 = 2 * g
        wait_linear(set0)
        shift_and_gather(set0)
        wait_linear(set1)
        shift_and_gather(set1)

        wait_gather(set0)
        drain_scatter(set0)
        compute(set0)
        start_scatter(set0)

        @pl.when(b0 + 2 < NBATCH)
        def _():
            issue_linear(b0 + 2, set0)

        wait_gather(set1)
        drain_scatter(set1)
        compute(set1)
        start_scatter(set1)

        @pl.when(b0 + 3 < NBATCH)
        def _():
            issue_linear(b0 + 3, set1)

        return carry

    lax.fori_loop(0, NBATCH // 2, pair_body, 0)
    drain_scatter(set0)
    drain_scatter(set1)
    plsc.subcore_barrier()
    pltpu.sync_copy(acc.at[pl.ds(rbase, ROWS_PER_SUB)],
                    out.at[c, pl.ds(rbase, ROWS_PER_SUB)])


def _sc_aggregate(tbl, src, dst, ea, q2):
    mesh = plsc.VectorSubcoreMesh(core_axis_name="c", subcore_axis_name="s")
    bufset = [
        pltpu.VMEM((EB,), jnp.int32),
        pltpu.VMEM((EB,), jnp.int32),
        pltpu.VMEM((EB, DE), jnp.float32),
        pltpu.VMEM((EB // 8, 128), jnp.float32),
        pltpu.VMEM((EB, TW), jnp.float32),
        pltpu.VMEM((EB, ACC_W), jnp.float32),
        pltpu.VMEM((EB,), jnp.int32),
    ]
    return pl.kernel(
        _sc_body,
        out_type=jax.ShapeDtypeStruct((NC, NPAD, ACC_W), jnp.float32),
        mesh=mesh,
        compiler_params=pltpu.CompilerParams(use_tc_tiling_on_sc=False),
        scratch_types=bufset + bufset + [
            pltpu.VMEM_SHARED((NPAD, ACC_W), jnp.float32),
            pltpu.SemaphoreType.DMA,
            pltpu.SemaphoreType.DMA,
            pltpu.SemaphoreType.DMA,
            pltpu.SemaphoreType.DMA,
            pltpu.SemaphoreType.DMA,
            pltpu.SemaphoreType.DMA,
        ],
    )(tbl, src, dst, ea, q2)


def _tbl_body(xf_ref, vm_ref, cst_ref, t_ref):
    cc = pl.program_id(0)
    xb = xf_ref[...]
    xc = jnp.where(cc == 0, xb[:, :2 * D], xb[:, 2 * D:])
    pa = (jnp.dot(xb, vm_ref[...],
                  preferred_element_type=jnp.float32)
          + cst_ref[...])
    p2 = jnp.where(cc == 0, pa[:, 0:2], pa[:, 2:4])
    zpad = jnp.zeros((xc.shape[0], TW - 2 * D - 2), jnp.float32)
    t_ref[...] = jnp.concatenate([xc, p2, zpad], axis=1)


def _build_table(x, vmat, cst):
    Bn = 1000
    return pl.pallas_call(
        _tbl_body,
        grid=(NC, N // Bn),
        in_specs=[
            pl.BlockSpec((Bn, K * D), lambda cc, i: (i, 0)),
            pl.BlockSpec((K * D, K), lambda cc, i: (0, 0)),
            pl.BlockSpec((1, K), lambda cc, i: (0, 0)),
        ],
        out_specs=pl.BlockSpec((Bn, TW),
                               lambda cc, i: (cc * (N // Bn) + i, 0)),
        out_shape=jax.ShapeDtypeStruct((NC * N, TW), jnp.float32),
    )(x, vmat, cst)


def _q_body(ea8_ref, bd_ref, q_ref):
    ea8 = ea8_ref[...]
    q_ref[0] = jnp.dot(ea8, bd_ref[0],
                       preferred_element_type=jnp.float32)
    q_ref[1] = jnp.dot(ea8, bd_ref[1],
                       preferred_element_type=jnp.float32)


def _build_q2(ea8, bd):
    Br = 1000
    R = E // 8
    return pl.pallas_call(
        _q_body,
        grid=(R // Br,),
        in_specs=[
            pl.BlockSpec((Br, 128), lambda i: (i, 0)),
            pl.BlockSpec((NC, 128, 128), lambda i: (0, 0, 0)),
        ],
        out_specs=pl.BlockSpec((NC, Br, 128), lambda i: (0, i, 0)),
        out_shape=jax.ShapeDtypeStruct((NC, R, 128), jnp.float32),
    )(ea8, bd)


def _epi_body(acc_ref, x_ref, af_ref, bf_ref, bl_ref, wr_ref,
              fw1_ref, fb1_ref, fw2_ref, out_ref):
    xb = x_ref[...]
    cnt = jnp.maximum(acc_ref[0, :, 98:99], 1.0)
    fW1 = fw1_ref[...]
    fb1 = fb1_ref[...]
    fW2 = fw2_ref[...]
    hs = []
    for k in range(K):
        c, j = divmod(k, 2)
        accc = acc_ref[c]
        sx = accc[:, j * D:(j + 1) * D]
        se = accc[:, 2 * D + j * DE:2 * D + (j + 1) * DE]
        sa = accc[:, 96 + j:97 + j]
        agg = (jnp.dot(sx, af_ref[k * D:(k + 1) * D, :],
                       preferred_element_type=jnp.float32)
               + jnp.dot(se, bf_ref[k * DE:(k + 1) * DE, :],
                         preferred_element_type=jnp.float32)
               + sa * bl_ref[k:k + 1, :]) / cnt
        hk = agg + jnp.dot(xb[:, k * D:(k + 1) * D],
                           wr_ref[k * D:(k + 1) * D, :],
                           preferred_element_type=jnp.float32)
        hs.append(hk)
    scores = []
    for k in range(K):
        zk = jnp.tanh(jnp.dot(hs[k], fW1, preferred_element_type=jnp.float32)
                      + fb1[0][None, :])
        scores.append(jnp.dot(zk, fW2, preferred_element_type=jnp.float32))
    sc = jnp.concatenate(scores, axis=1)
    sc = sc - jnp.max(sc, axis=1, keepdims=True)
    es = jnp.exp(sc)
    w = es / jnp.sum(es, axis=1, keepdims=True)
    o = jnp.zeros_like(hs[0])
    for k in range(K):
        o = o + w[:, k][:, None] * hs[k]
    out_ref[...] = o


def _epilogue(accs, x, af, bf, blm, wrf, fW1, fb1, fW2):
    Bn = 1000
    return pl.pallas_call(
        _epi_body,
        grid=(N // Bn,),
        in_specs=[
            pl.BlockSpec((NC, Bn, ACC_W), lambda i: (0, i, 0)),
            pl.BlockSpec((Bn, K * D), lambda i: (i, 0)),
            pl.BlockSpec((K * D, H), lambda i: (0, 0)),
            pl.BlockSpec((K * DE, H), lambda i: (0, 0)),
            pl.BlockSpec((K, H), lambda i: (0, 0)),
            pl.BlockSpec((K * D, H), lambda i: (0, 0)),
            pl.BlockSpec((H, H), lambda i: (0, 0)),
            pl.BlockSpec((1, H), lambda i: (0, 0)),
            pl.BlockSpec((H, 1), lambda i: (0, 0)),
        ],
        out_specs=pl.BlockSpec((Bn, H), lambda i: (i, 0)),
        out_shape=jax.ShapeDtypeStruct((N, H), jnp.float32),
    )(accs, x, af, bf, blm, wrf, fW1, fb1, fW2)


def kernel(x, edge_index, edge_attr, Wl, bl, Wr, aw, ab, fW1, fb1, fW2):
    src = edge_index[0]
    dst = edge_index[1]
    A = Wl[:, :D, :]
    B = Wl[:, D:, :]
    awv = aw[..., 0]
    v = jnp.einsum('kdh,kh->kd', A, awv)
    cst = (jnp.einsum('kh,kh->k', bl, awv) + ab[:, 0]).reshape(1, K)
    vmat = jax.scipy.linalg.block_diag(
        *[v[k][:, None] for k in range(K)])
    wq = jnp.einsum('kdh,kh->kd', B, awv).T
    bd = jnp.stack([
        jnp.kron(jnp.eye(8, dtype=jnp.float32),
                 jnp.pad(wq[:, 2 * cc:2 * cc + 2], ((0, 0), (0, 14))))
        for cc in range(NC)])
    af = A.reshape(K * D, H)
    bf = B.reshape(K * DE, H)
    wrf = Wr.reshape(K * D, H)

    tbl = _build_table(x, vmat, cst)
    q2 = _build_q2(edge_attr.reshape(E // 8, 128), bd)
    accs = _sc_aggregate(tbl, src, dst, edge_attr, q2)
    return _epilogue(accs, x, af, bf, bl, wrf, fW1,
                     fb1.reshape(1, H), fW2)


# speedup vs baseline: 1.0074x; 1.0074x over previous
import jax
import jax.numpy as jnp
from jax import lax
from jax.experimental import pallas as pl
from jax.experimental.pallas import tpu as pltpu
from jax.experimental.pallas import tpu_sc as plsc

N = 10000
E = 320000
D = 32
DE = 16
H = 64
K = 4

NC = 2
NS = 16
NPAD = 10240
ROWS_PER_SUB = NPAD // NS
ACC_W = 128
TW = 128
EB = 80
EDGES_PER_SUB = E // NS
NBATCH = EDGES_PER_SUB // EB


def _sc_body(tbl, srcA, dstA, eaA, q2A, out,
             idx0, dst0, ea0, q20, g0, c0, ds0,
             idx1, dst1, ea1, q21, g1, c1, ds1,
             acc, sl0, sg0, ss0, sl1, sg1, ss1):
    c = lax.axis_index("c")
    s = lax.axis_index("s")
    set0 = (idx0, dst0, ea0, q20, g0, c0, ds0, sl0, sg0, ss0)
    set1 = (idx1, dst1, ea1, q21, g1, c1, ds1, sl1, sg1, ss1)

    zf = jnp.zeros((16,), jnp.float32)
    zi = jnp.zeros((16,), jnp.int32)

    def zrow_body(i, cc):
        for col in range(0, ACC_W, 16):
            c0[i, pl.ds(col, 16)] = zf
            c1[i, pl.ds(col, 16)] = zf
        return cc

    lax.fori_loop(0, EB, zrow_body, 0)
    for j in range(EB // 16):
        ds0[pl.ds(j * 16, 16)] = zi
        ds1[pl.ds(j * 16, 16)] = zi
    rbase = s * ROWS_PER_SUB
    for i in range(ROWS_PER_SUB // EB):
        pltpu.sync_copy(c0, acc.at[pl.ds(rbase + i * EB, EB)])
    plsc.subcore_barrier()

    ebase = s * EDGES_PER_SUB

    def issue_linear(b, S):
        idx_v, dst_v, ea_v, q2_v = S[0], S[1], S[2], S[3]
        sem_l = S[7]
        base = ebase + b * EB
        pltpu.async_copy(srcA.at[pl.ds(base, EB)], idx_v, sem_l)
        pltpu.async_copy(dstA.at[pl.ds(base, EB)], dst_v, sem_l)
        pltpu.async_copy(eaA.at[pl.ds(base, EB)], ea_v, sem_l)
        qbase = s * (EDGES_PER_SUB // 8) + b * (EB // 8)
        pltpu.async_copy(q2A.at[c, pl.ds(qbase, EB // 8)], q2_v, sem_l)

    def wait_linear(S):
        idx_v, dst_v, ea_v, q2_v = S[0], S[1], S[2], S[3]
        sem_l = S[7]
        pltpu.make_async_copy(srcA.at[pl.ds(0, EB)], idx_v, sem_l).wait()
        pltpu.make_async_copy(dstA.at[pl.ds(0, EB)], dst_v, sem_l).wait()
        pltpu.make_async_copy(eaA.at[pl.ds(0, EB)], ea_v, sem_l).wait()
        pltpu.make_async_copy(q2A.at[c, pl.ds(0, EB // 8)], q2_v,
                              sem_l).wait()

    def shift_and_gather(S):
        idx_v, g_v, sem_g = S[0], S[4], S[8]
        coff_v = jnp.full((16,), c * N, jnp.int32)
        for j in range(EB // 16):
            sl = pl.ds(j * 16, 16)
            idx_v[sl] = idx_v[sl] + coff_v
        pltpu.async_copy(tbl.at[idx_v], g_v, sem_g)

    def wait_gather(S):
        idx_v, g_v, sem_g = S[0], S[4], S[8]
        pltpu.make_async_copy(tbl.at[idx_v], g_v, sem_g).wait()

    def drain_scatter(S):
        c_v, dstS, sem_s = S[5], S[6], S[9]
        pltpu.make_async_copy(c_v, acc.at[dstS], sem_s).wait()

    def start_scatter(S):
        dst_v, c_v, dstS, sem_s = S[1], S[5], S[6], S[9]
        for j in range(EB // 16):
            sl = pl.ds(j * 16, 16)
            dstS[sl] = dst_v[sl]
        pltpu.async_copy(c_v, acc.at[dstS], sem_s, add=True)

    def compute(S):
        ea_v, q2_v, g_v, c_v = S[2], S[3], S[4], S[5]

        @plsc.parallel_loop(0, EB, 1, unroll=4)
        def edge_body(e):
            lanesf = jnp.arange(16, dtype=jnp.int32).astype(jnp.float32)
            m01f = jnp.maximum(0.0, jnp.minimum(1.0, 2.0 - lanesf))
            oh2c = jnp.maximum(0.0, 1.0 - jnp.abs(lanesf - 2.0))
            t = (g_v[e, pl.ds(64, 16)]
                 + q2_v[lax.shift_right_logical(e, 3),
                        pl.ds(lax.bitwise_and(e, 7) * 16, 16)])
            e2 = jnp.exp(t + t)
            ar = 1.0 - 2.0 / (e2 + 1.0)
            a0 = ar[0]
            a1 = ar[1]
            c_v[e, pl.ds(0, 16)] = a0 * g_v[e, pl.ds(0, 16)]
            c_v[e, pl.ds(16, 16)] = a0 * g_v[e, pl.ds(16, 16)]
            c_v[e, pl.ds(32, 16)] = a1 * g_v[e, pl.ds(32, 16)]
            c_v[e, pl.ds(48, 16)] = a1 * g_v[e, pl.ds(48, 16)]
            ear = ea_v[e, :]
            c_v[e, pl.ds(64, 16)] = a0 * ear
            c_v[e, pl.ds(80, 16)] = a1 * ear
            c_v[e, pl.ds(96, 16)] = ar * m01f + oh2c

    pltpu.async_copy(c0, acc.at[ds0], ss0, add=True)
    pltpu.async_copy(c1, acc.at[ds1], ss1, add=True)
    issue_linear(0, set0)
    issue_linear(1, set1)

    def pair_body(g, carry):
        b0 = 2 * g
        wait_linear(set0)
        shift_and_gather(set0)
        wait_linear(set1)
        shift_and_gather(set1)

        wait_gather(set0)
        drain_scatter(set0)
        compute(set0)
        start_scatter(set0)

        @pl.when(b0 + 2 < NBATCH)
        def _():
            issue_linear(b0 + 2, set0)

        wait_gather(set1)
        drain_scatter(set1)
        compute(set1)
        start_scatter(set1)

        @pl.when(b0 + 3 < NBATCH)
        def _():
            issue_linear(b0 + 3, set1)

        return carry

    lax.fori_loop(0, NBATCH // 2, pair_body, 0)
    drain_scatter(set0)
    drain_scatter(set1)
    plsc.subcore_barrier()
    pltpu.sync_copy(acc.at[pl.ds(rbase, ROWS_PER_SUB)],
                    out.at[c, pl.ds(rbase, ROWS_PER_SUB)])


def _sc_aggregate(tbl, src, dst, ea, q2):
    mesh = plsc.VectorSubcoreMesh(core_axis_name="c", subcore_axis_name="s")
    bufset = [
        pltpu.VMEM((EB,), jnp.int32),
        pltpu.VMEM((EB,), jnp.int32),
        pltpu.VMEM((EB, DE), jnp.float32),
        pltpu.VMEM((EB // 8, 128), jnp.float32),
        pltpu.VMEM((EB, TW), jnp.float32),
        pltpu.VMEM((EB, ACC_W), jnp.float32),
        pltpu.VMEM((EB,), jnp.int32),
    ]
    return pl.kernel(
        _sc_body,
        out_type=jax.ShapeDtypeStruct((NC, NPAD, ACC_W), jnp.float32),
        mesh=mesh,
        compiler_params=pltpu.CompilerParams(use_tc_tiling_on_sc=False),
        scratch_types=bufset + bufset + [
            pltpu.VMEM_SHARED((NPAD, ACC_W), jnp.float32),
            pltpu.SemaphoreType.DMA,
            pltpu.SemaphoreType.DMA,
            pltpu.SemaphoreType.DMA,
            pltpu.SemaphoreType.DMA,
            pltpu.SemaphoreType.DMA,
            pltpu.SemaphoreType.DMA,
        ],
    )(tbl, src, dst, ea, q2)


def _tbl_body(x_ref, vm_ref, cst_ref, t_ref):
    xb = x_ref[...]
    p = (jnp.dot(xb, vm_ref[...], preferred_element_type=jnp.float32)
         + cst_ref[...])
    zpad = jnp.zeros((xb.shape[0], TW - 2 * D - 2), jnp.float32)
    t_ref[0] = jnp.concatenate([xb[:, :2 * D], p[:, 0:2], zpad], axis=1)
    t_ref[1] = jnp.concatenate([xb[:, 2 * D:], p[:, 2:4], zpad], axis=1)


def _build_table(x, vmat, cst):
    Bn = 1000
    return pl.pallas_call(
        _tbl_body,
        grid=(N // Bn,),
        in_specs=[
            pl.BlockSpec((Bn, K * D), lambda i: (i, 0)),
            pl.BlockSpec((K * D, K), lambda i: (0, 0)),
            pl.BlockSpec((1, K), lambda i: (0, 0)),
        ],
        out_specs=pl.BlockSpec((NC, Bn, TW), lambda i: (0, i, 0)),
        out_shape=jax.ShapeDtypeStruct((NC, N, TW), jnp.float32),
    )(x, vmat, cst)


def _q_body(ea8_ref, bd_ref, q_ref):
    ea8 = ea8_ref[...]
    q_ref[0] = jnp.dot(ea8, bd_ref[0],
                       preferred_element_type=jnp.float32)
    q_ref[1] = jnp.dot(ea8, bd_ref[1],
                       preferred_element_type=jnp.float32)


def _build_q2(ea8, bd):
    Br = 1000
    R = E // 8
    return pl.pallas_call(
        _q_body,
        grid=(R // Br,),
        in_specs=[
            pl.BlockSpec((Br, 128), lambda i: (i, 0)),
            pl.BlockSpec((NC, 128, 128), lambda i: (0, 0, 0)),
        ],
        out_specs=pl.BlockSpec((NC, Br, 128), lambda i: (0, i, 0)),
        out_shape=jax.ShapeDtypeStruct((NC, R, 128), jnp.float32),
    )(ea8, bd)


def _epi_body(acc_ref, x_ref, af_ref, bf_ref, bl_ref, wr_ref,
              fw1_ref, fb1_ref, fw2_ref, out_ref):
    xb = x_ref[...]
    cnt = jnp.maximum(acc_ref[0, :, 98:99], 1.0)
    fW1 = fw1_ref[...]
    fb1 = fb1_ref[...]
    fW2 = fw2_ref[...]
    hs = []
    for k in range(K):
        c, j = divmod(k, 2)
        accc = acc_ref[c]
        sx = accc[:, j * D:(j + 1) * D]
        se = accc[:, 2 * D + j * DE:2 * D + (j + 1) * DE]
        sa = accc[:, 96 + j:97 + j]
        agg = (jnp.dot(sx, af_ref[k * D:(k + 1) * D, :],
                       preferred_element_type=jnp.float32)
               + jnp.dot(se, bf_ref[k * DE:(k + 1) * DE, :],
                         preferred_element_type=jnp.float32)
               + sa * bl_ref[k:k + 1, :]) / cnt
        hk = agg + jnp.dot(xb[:, k * D:(k + 1) * D],
                           wr_ref[k * D:(k + 1) * D, :],
                           preferred_element_type=jnp.float32)
        hs.append(hk)
    scores = []
    for k in range(K):
        zk = jnp.tanh(jnp.dot(hs[k], fW1, preferred_element_type=jnp.float32)
                      + fb1[0][None, :])
        scores.append(jnp.dot(zk, fW2, preferred_element_type=jnp.float32))
    sc = jnp.concatenate(scores, axis=1)
    sc = sc - jnp.max(sc, axis=1, keepdims=True)
    es = jnp.exp(sc)
    w = es / jnp.sum(es, axis=1, keepdims=True)
    o = jnp.zeros_like(hs[0])
    for k in range(K):
        o = o + w[:, k][:, None] * hs[k]
    out_ref[...] = o


def _epilogue(accs, x, af, bf, blm, wrf, fW1, fb1, fW2):
    Bn = 1000
    return pl.pallas_call(
        _epi_body,
        grid=(N // Bn,),
        in_specs=[
            pl.BlockSpec((NC, Bn, ACC_W), lambda i: (0, i, 0)),
            pl.BlockSpec((Bn, K * D), lambda i: (i, 0)),
            pl.BlockSpec((K * D, H), lambda i: (0, 0)),
            pl.BlockSpec((K * DE, H), lambda i: (0, 0)),
            pl.BlockSpec((K, H), lambda i: (0, 0)),
            pl.BlockSpec((K * D, H), lambda i: (0, 0)),
            pl.BlockSpec((H, H), lambda i: (0, 0)),
            pl.BlockSpec((1, H), lambda i: (0, 0)),
            pl.BlockSpec((H, 1), lambda i: (0, 0)),
        ],
        out_specs=pl.BlockSpec((Bn, H), lambda i: (i, 0)),
        out_shape=jax.ShapeDtypeStruct((N, H), jnp.float32),
    )(accs, x, af, bf, blm, wrf, fW1, fb1, fW2)


def kernel(x, edge_index, edge_attr, Wl, bl, Wr, aw, ab, fW1, fb1, fW2):
    src = edge_index[0]
    dst = edge_index[1]
    A = Wl[:, :D, :]
    B = Wl[:, D:, :]
    awv = aw[..., 0]
    v = jnp.einsum('kdh,kh->kd', A, awv)
    cst = (jnp.einsum('kh,kh->k', bl, awv) + ab[:, 0]).reshape(1, K)
    vmat = jax.scipy.linalg.block_diag(
        *[v[k][:, None] for k in range(K)])
    wq = jnp.einsum('kdh,kh->kd', B, awv).T
    bd = jnp.stack([
        jnp.kron(jnp.eye(8, dtype=jnp.float32),
                 jnp.pad(wq[:, 2 * cc:2 * cc + 2], ((0, 0), (0, 14))))
        for cc in range(NC)])
    af = A.reshape(K * D, H)
    bf = B.reshape(K * DE, H)
    wrf = Wr.reshape(K * D, H)

    tbl = _build_table(x, vmat, cst).reshape(NC * N, TW)
    q2 = _build_q2(edge_attr.reshape(E // 8, 128), bd)
    accs = _sc_aggregate(tbl, src, dst, edge_attr, q2)
    return _epilogue(accs, x, af, bf, bl, wrf, fW1,
                     fb1.reshape(1, H), fW2)


# speedup vs baseline: 1.0151x; 1.0076x over previous
import jax
import jax.numpy as jnp
from jax import lax
from jax.experimental import pallas as pl
from jax.experimental.pallas import tpu as pltpu
from jax.experimental.pallas import tpu_sc as plsc

N = 10000
E = 320000
D = 32
DE = 16
H = 64
K = 4

NC = 2
NS = 16
NPAD = 10240
ROWS_PER_SUB = NPAD // NS
ACC_W = 128
TW = 128
EB = 80
EDGES_PER_SUB = E // NS
NBATCH = EDGES_PER_SUB // EB


def _sc_body(tbl, srcA, dstA, eaA, q2A, out,
             idx0, dst0, ea0, q20, g0, c0, ds0,
             idx1, dst1, ea1, q21, g1, c1, ds1,
             acc, sl0, sg0, ss0, sl1, sg1, ss1):
    c = lax.axis_index("c")
    s = lax.axis_index("s")
    set0 = (idx0, dst0, ea0, q20, g0, c0, ds0, sl0, sg0, ss0)
    set1 = (idx1, dst1, ea1, q21, g1, c1, ds1, sl1, sg1, ss1)

    zf = jnp.zeros((16,), jnp.float32)
    zi = jnp.zeros((16,), jnp.int32)

    def zrow_body(i, cc):
        for col in range(0, ACC_W, 16):
            c0[i, pl.ds(col, 16)] = zf
            c1[i, pl.ds(col, 16)] = zf
        return cc

    lax.fori_loop(0, EB, zrow_body, 0)
    for j in range(EB // 16):
        ds0[pl.ds(j * 16, 16)] = zi
        ds1[pl.ds(j * 16, 16)] = zi
    rbase = s * ROWS_PER_SUB
    for i in range(ROWS_PER_SUB // EB):
        pltpu.sync_copy(c0, acc.at[pl.ds(rbase + i * EB, EB)])
    plsc.subcore_barrier()

    ebase = s * EDGES_PER_SUB

    def issue_linear(b, S):
        idx_v, dst_v, ea_v, q2_v = S[0], S[1], S[2], S[3]
        sem_l = S[7]
        base = ebase + b * EB
        pltpu.async_copy(srcA.at[pl.ds(base, EB)], idx_v, sem_l)
        pltpu.async_copy(dstA.at[pl.ds(base, EB)], dst_v, sem_l)
        pltpu.async_copy(eaA.at[pl.ds(base, EB)], ea_v, sem_l)
        qbase = s * (EDGES_PER_SUB // 8) + b * (EB // 8)
        pltpu.async_copy(q2A.at[c, pl.ds(qbase, EB // 8)], q2_v, sem_l)

    def wait_linear(S):
        idx_v, dst_v, ea_v, q2_v = S[0], S[1], S[2], S[3]
        sem_l = S[7]
        pltpu.make_async_copy(srcA.at[pl.ds(0, EB)], idx_v, sem_l).wait()
        pltpu.make_async_copy(dstA.at[pl.ds(0, EB)], dst_v, sem_l).wait()
        pltpu.make_async_copy(eaA.at[pl.ds(0, EB)], ea_v, sem_l).wait()
        pltpu.make_async_copy(q2A.at[c, pl.ds(0, EB // 8)], q2_v,
                              sem_l).wait()

    def shift_and_gather(S):
        idx_v, g_v, sem_g = S[0], S[4], S[8]
        coff_v = jnp.full((16,), c * N, jnp.int32)
        for j in range(EB // 16):
            sl = pl.ds(j * 16, 16)
            idx_v[sl] = idx_v[sl] + coff_v
        pltpu.async_copy(tbl.at[idx_v], g_v, sem_g)

    def wait_gather(S):
        idx_v, g_v, sem_g = S[0], S[4], S[8]
        pltpu.make_async_copy(tbl.at[idx_v], g_v, sem_g).wait()

    def drain_scatter(S):
        c_v, dstS, sem_s = S[5], S[6], S[9]
        pltpu.make_async_copy(c_v, acc.at[dstS], sem_s).wait()

    def start_scatter(S):
        dst_v, c_v, dstS, sem_s = S[1], S[5], S[6], S[9]
        for j in range(EB // 16):
            sl = pl.ds(j * 16, 16)
            dstS[sl] = dst_v[sl]
        pltpu.async_copy(c_v, acc.at[dstS], sem_s, add=True)

    def compute(S):
        ea_v, q2_v, g_v, c_v = S[2], S[3], S[4], S[5]

        @plsc.parallel_loop(0, EB, 1, unroll=2)
        def edge_body(e):
            lanesf = jnp.arange(16, dtype=jnp.int32).astype(jnp.float32)
            m01f = jnp.maximum(0.0, jnp.minimum(1.0, 2.0 - lanesf))
            oh2c = jnp.maximum(0.0, 1.0 - jnp.abs(lanesf - 2.0))
            t = (g_v[e, pl.ds(64, 16)]
                 + q2_v[lax.shift_right_logical(e, 3),
                        pl.ds(lax.bitwise_and(e, 7) * 16, 16)])
            e2 = jnp.exp(t + t)
            ar = 1.0 - 2.0 / (e2 + 1.0)
            a0 = ar[0]
            a1 = ar[1]
            c_v[e, pl.ds(0, 16)] = a0 * g_v[e, pl.ds(0, 16)]
            c_v[e, pl.ds(16, 16)] = a0 * g_v[e, pl.ds(16, 16)]
            c_v[e, pl.ds(32, 16)] = a1 * g_v[e, pl.ds(32, 16)]
            c_v[e, pl.ds(48, 16)] = a1 * g_v[e, pl.ds(48, 16)]
            ear = ea_v[e, :]
            c_v[e, pl.ds(64, 16)] = a0 * ear
            c_v[e, pl.ds(80, 16)] = a1 * ear
            c_v[e, pl.ds(96, 16)] = ar * m01f + oh2c

    pltpu.async_copy(c0, acc.at[ds0], ss0, add=True)
    pltpu.async_copy(c1, acc.at[ds1], ss1, add=True)
    issue_linear(0, set0)
    issue_linear(1, set1)

    def pair_body(g, carry):
        b0 = 2 * g
        wait_linear(set0)
        shift_and_gather(set0)
        wait_linear(set1)
        shift_and_gather(set1)

        wait_gather(set0)
        drain_scatter(set0)
        compute(set0)
        start_scatter(set0)

        @pl.when(b0 + 2 < NBATCH)
        def _():
            issue_linear(b0 + 2, set0)

        wait_gather(set1)
        drain_scatter(set1)
        compute(set1)
        start_scatter(set1)

        @pl.when(b0 + 3 < NBATCH)
        def _():
            issue_linear(b0 + 3, set1)

        return carry

    lax.fori_loop(0, NBATCH // 2, pair_body, 0)
    drain_scatter(set0)
    drain_scatter(set1)
    plsc.subcore_barrier()
    pltpu.sync_copy(acc.at[pl.ds(rbase, ROWS_PER_SUB)],
                    out.at[c, pl.ds(rbase, ROWS_PER_SUB)])


def _sc_aggregate(tbl, src, dst, ea, q2):
    mesh = plsc.VectorSubcoreMesh(core_axis_name="c", subcore_axis_name="s")
    bufset = [
        pltpu.VMEM((EB,), jnp.int32),
        pltpu.VMEM((EB,), jnp.int32),
        pltpu.VMEM((EB, DE), jnp.float32),
        pltpu.VMEM((EB // 8, 128), jnp.float32),
        pltpu.VMEM((EB, TW), jnp.float32),
        pltpu.VMEM((EB, ACC_W), jnp.float32),
        pltpu.VMEM((EB,), jnp.int32),
    ]
    return pl.kernel(
        _sc_body,
        out_type=jax.ShapeDtypeStruct((NC, NPAD, ACC_W), jnp.float32),
        mesh=mesh,
        compiler_params=pltpu.CompilerParams(use_tc_tiling_on_sc=False),
        scratch_types=bufset + bufset + [
            pltpu.VMEM_SHARED((NPAD, ACC_W), jnp.float32),
            pltpu.SemaphoreType.DMA,
            pltpu.SemaphoreType.DMA,
            pltpu.SemaphoreType.DMA,
            pltpu.SemaphoreType.DMA,
            pltpu.SemaphoreType.DMA,
            pltpu.SemaphoreType.DMA,
        ],
    )(tbl, src, dst, ea, q2)


def _tbl_body(x_ref, vm_ref, cst_ref, t_ref):
    xb = x_ref[...]
    p = (jnp.dot(xb, vm_ref[...], preferred_element_type=jnp.float32)
         + cst_ref[...])
    zpad = jnp.zeros((xb.shape[0], TW - 2 * D - 2), jnp.float32)
    t_ref[0] = jnp.concatenate([xb[:, :2 * D], p[:, 0:2], zpad], axis=1)
    t_ref[1] = jnp.concatenate([xb[:, 2 * D:], p[:, 2:4], zpad], axis=1)


def _build_table(x, vmat, cst):
    Bn = 1000
    return pl.pallas_call(
        _tbl_body,
        grid=(N // Bn,),
        in_specs=[
            pl.BlockSpec((Bn, K * D), lambda i: (i, 0)),
            pl.BlockSpec((K * D, K), lambda i: (0, 0)),
            pl.BlockSpec((1, K), lambda i: (0, 0)),
        ],
        out_specs=pl.BlockSpec((NC, Bn, TW), lambda i: (0, i, 0)),
        out_shape=jax.ShapeDtypeStruct((NC, N, TW), jnp.float32),
    )(x, vmat, cst)


def _q_body(ea8_ref, bd_ref, q_ref):
    ea8 = ea8_ref[...]
    q_ref[0] = jnp.dot(ea8, bd_ref[0],
                       preferred_element_type=jnp.float32)
    q_ref[1] = jnp.dot(ea8, bd_ref[1],
                       preferred_element_type=jnp.float32)


def _build_q2(ea8, bd):
    Br = 1000
    R = E // 8
    return pl.pallas_call(
        _q_body,
        grid=(R // Br,),
        in_specs=[
            pl.BlockSpec((Br, 128), lambda i: (i, 0)),
            pl.BlockSpec((NC, 128, 128), lambda i: (0, 0, 0)),
        ],
        out_specs=pl.BlockSpec((NC, Br, 128), lambda i: (0, i, 0)),
        out_shape=jax.ShapeDtypeStruct((NC, R, 128), jnp.float32),
    )(ea8, bd)


def _epi_body(acc_ref, x_ref, af_ref, bf_ref, bl_ref, wr_ref,
              fw1_ref, fb1_ref, fw2_ref, out_ref):
    xb = x_ref[...]
    cnt = jnp.maximum(acc_ref[0, :, 98:99], 1.0)
    fW1 = fw1_ref[...]
    fb1 = fb1_ref[...]
    fW2 = fw2_ref[...]
    hs = []
    for k in range(K):
        c, j = divmod(k, 2)
        accc = acc_ref[c]
        sx = accc[:, j * D:(j + 1) * D]
        se = accc[:, 2 * D + j * DE:2 * D + (j + 1) * DE]
        sa = accc[:, 96 + j:97 + j]
        agg = (jnp.dot(sx, af_ref[k * D:(k + 1) * D, :],
                       preferred_element_type=jnp.float32)
               + jnp.dot(se, bf_ref[k * DE:(k + 1) * DE, :],
                         preferred_element_type=jnp.float32)
               + sa * bl_ref[k:k + 1, :]) / cnt
        hk = agg + jnp.dot(xb[:, k * D:(k + 1) * D],
                           wr_ref[k * D:(k + 1) * D, :],
                           preferred_element_type=jnp.float32)
        hs.append(hk)
    scores = []
    for k in range(K):
        zk = jnp.tanh(jnp.dot(hs[k], fW1, preferred_element_type=jnp.float32)
                      + fb1[0][None, :])
        scores.append(jnp.dot(zk, fW2, preferred_element_type=jnp.float32))
    sc = jnp.concatenate(scores, axis=1)
    sc = sc - jnp.max(sc, axis=1, keepdims=True)
    es = jnp.exp(sc)
    w = es / jnp.sum(es, axis=1, keepdims=True)
    o = jnp.zeros_like(hs[0])
    for k in range(K):
        o = o + w[:, k][:, None] * hs[k]
    out_ref[...] = o


def _epilogue(accs, x, af, bf, blm, wrf, fW1, fb1, fW2):
    Bn = 1000
    return pl.pallas_call(
        _epi_body,
        grid=(N // Bn,),
        in_specs=[
            pl.BlockSpec((NC, Bn, ACC_W), lambda i: (0, i, 0)),
            pl.BlockSpec((Bn, K * D), lambda i: (i, 0)),
            pl.BlockSpec((K * D, H), lambda i: (0, 0)),
            pl.BlockSpec((K * DE, H), lambda i: (0, 0)),
            pl.BlockSpec((K, H), lambda i: (0, 0)),
            pl.BlockSpec((K * D, H), lambda i: (0, 0)),
            pl.BlockSpec((H, H), lambda i: (0, 0)),
            pl.BlockSpec((1, H), lambda i: (0, 0)),
            pl.BlockSpec((H, 1), lambda i: (0, 0)),
        ],
        out_specs=pl.BlockSpec((Bn, H), lambda i: (i, 0)),
        out_shape=jax.ShapeDtypeStruct((N, H), jnp.float32),
    )(accs, x, af, bf, blm, wrf, fW1, fb1, fW2)


def kernel(x, edge_index, edge_attr, Wl, bl, Wr, aw, ab, fW1, fb1, fW2):
    src = edge_index[0]
    dst = edge_index[1]
    A = Wl[:, :D, :]
    B = Wl[:, D:, :]
    awv = aw[..., 0]
    v = jnp.einsum('kdh,kh->kd', A, awv)
    cst = (jnp.einsum('kh,kh->k', bl, awv) + ab[:, 0]).reshape(1, K)
    vmat = jax.scipy.linalg.block_diag(
        *[v[k][:, None] for k in range(K)])
    wq = jnp.einsum('kdh,kh->kd', B, awv).T
    bd = jnp.stack([
        jnp.kron(jnp.eye(8, dtype=jnp.float32),
                 jnp.pad(wq[:, 2 * cc:2 * cc + 2], ((0, 0), (0, 14))))
        for cc in range(NC)])
    af = A.reshape(K * D, H)
    bf = B.reshape(K * DE, H)
    wrf = Wr.reshape(K * D, H)

    tbl = _build_table(x, vmat, cst).reshape(NC * N, TW)
    q2 = _build_q2(edge_attr.reshape(E // 8, 128), bd)
    accs = _sc_aggregate(tbl, src, dst, edge_attr, q2)
    return _epilogue(accs, x, af, bf, bl, wrf, fW1,
                     fb1.reshape(1, H), fW2)
